# Initial kernel scaffold; baseline (speedup 1.0000x reference)
import jax
import jax.numpy as jnp
from jax.experimental import pallas as pl


def kernel(inputs, ex_inputs, edge_index, edge_weight, W1, b1, W2, b2, g1, be1, g2, be2, fc1_W, fc1_b, fc2_W, fc2_b):
    raise NotImplementedError("write your pallas kernel here")


# speedup vs baseline: 53.5295x; 53.5295x over previous
import functools

import jax
import jax.numpy as jnp
from jax import lax
from jax.experimental import pallas as pl
from jax.experimental.pallas import tpu as pltpu
from jax.experimental.pallas import tpu_sc as plsc

_M = 2048
_B = 64
_HID = 32
_E = 65536
_FC1 = 512
_EPS = 1e-5
_BH = _B * _HID
_N = float(_M * _B)

_RB = 128


def _dot3(a, b):
    f32 = jnp.float32
    a_hi = a.astype(jnp.bfloat16)
    a_lo = (a - a_hi.astype(f32)).astype(jnp.bfloat16)
    b_hi = b.astype(jnp.bfloat16)
    b_lo = (b - b_hi.astype(f32)).astype(jnp.bfloat16)
    return (jnp.dot(a_hi, b_hi, preferred_element_type=f32)
            + jnp.dot(a_hi, b_lo, preferred_element_type=f32)
            + jnp.dot(a_lo, b_hi, preferred_element_type=f32))

_NC, _NS = 2, 16
_NW = _NC * _NS
_RP = 32
_PASSES = _M // (_NW * _RP)
_CH = 2048


def _l_build_body(dst_hbm, src_hbm, w_hbm, zeros_hbm, l_hbm,
                  lblk, dstv, srcv, wv):
    wid = lax.axis_index("s") * _NC + lax.axis_index("c")
    for p in range(_PASSES):
        rbase = (wid * _PASSES + p) * _RP
        pltpu.sync_copy(zeros_hbm, lblk)

        def chunk(c, _, rbase=rbase):
            base = c * _CH
            pltpu.sync_copy(dst_hbm.at[pl.ds(base, _CH)], dstv)
            pltpu.sync_copy(src_hbm.at[pl.ds(base, _CH)], srcv)
            pltpu.sync_copy(w_hbm.at[pl.ds(base, _CH)], wv)

            def vec(i, __):
                off = i * 16
                d = dstv[pl.ds(off, 16)]
                s = srcv[pl.ds(off, 16)]
                w = wv[pl.ds(off, 16)]
                rel = d - rbase
                msk = (rel >= 0) & (rel < _RP)
                flat = rel * _M + s
                plsc.addupdate_scatter(lblk, [flat], w, mask=msk)
                return 0

            return lax.fori_loop(0, _CH // 16, vec, 0)

        lax.fori_loop(0, _E // _CH, chunk, 0)
        pltpu.sync_copy(lblk, l_hbm.at[pl.ds(rbase * _M, _RP * _M)])


def _build_l(edge_dst, edge_src, edge_w, zeros):
    fn = pl.kernel(
        _l_build_body,
        out_type=jax.ShapeDtypeStruct((_M * _M,), jnp.float32),
        mesh=plsc.VectorSubcoreMesh(core_axis_name="c", subcore_axis_name="s"),
        compiler_params=pltpu.CompilerParams(needs_layout_passes=False),
        scratch_types=[
            pltpu.VMEM((_RP * _M,), jnp.float32),
            pltpu.VMEM((_CH,), jnp.int32),
            pltpu.VMEM((_CH,), jnp.int32),
            pltpu.VMEM((_CH,), jnp.float32),
        ],
    )
    return fn(edge_dst, edge_src, edge_w, zeros)


def _xw1_body(x_ref, w_ref, out_ref):
    for i in range(4):
        out_ref[:, i * _HID:(i + 1) * _HID] = _dot3(x_ref[i], w_ref[...])


def _xw1(inputs, W1):
    w_in = W1.shape[0]
    return pl.pallas_call(
        _xw1_body,
        grid=(_B // 4,),
        in_specs=[
            pl.BlockSpec((4, _M, w_in), lambda g: (g, 0, 0)),
            pl.BlockSpec((w_in, _HID), lambda g: (0, 0)),
        ],
        out_specs=pl.BlockSpec((_M, 4 * _HID), lambda g: (0, g)),
        out_shape=jax.ShapeDtypeStruct((_M, _BH), jnp.float32),
    )(inputs, W1)


def _sel_fold():
    ch = lax.broadcasted_iota(jnp.int32, (_BH, _HID), 0) % _HID
    kk = lax.broadcasted_iota(jnp.int32, (_BH, _HID), 1)
    return (ch == kk).astype(jnp.float32)


def _sel_bcast(width):
    ch = lax.broadcasted_iota(jnp.int32, (_HID, width), 1) % _HID
    kk = lax.broadcasted_iota(jnp.int32, (_HID, width), 0)
    return (ch == kk).astype(jnp.float32)


def _scale_shift(ssum, ssq, gamma, beta):
    mean = ssum / _N
    var = ssq / _N - mean * mean
    scale = lax.rsqrt(var + _EPS) * gamma
    shift = beta - mean * scale
    return scale, shift


def _spmm1_body(l_ref, x_ref, b_ref, out_ref, st_ref, acc_ref):
    k = pl.program_id(0)

    @pl.when(k == 0)
    def _():
        acc_ref[...] = jnp.zeros_like(acc_ref)

    t = _dot3(l_ref[...], x_ref[...]) + b_ref[...]
    out_ref[...] = t
    acc_ref[0:1, :] += jnp.sum(t, axis=0, keepdims=True)
    acc_ref[1:2, :] += jnp.sum(t * t, axis=0, keepdims=True)

    @pl.when(k == pl.num_programs(0) - 1)
    def _():
        st_ref[...] = acc_ref[...]


def _spmm1(L, xw1, b1t):
    return pl.pallas_call(
        _spmm1_body,
        grid=(_M // _RB,),
        in_specs=[
            pl.BlockSpec((_RB, _M), lambda k: (k, 0)),
            pl.BlockSpec((_M, _BH), lambda k: (0, 0)),
            pl.BlockSpec((1, _BH), lambda k: (0, 0)),
        ],
        out_specs=[
            pl.BlockSpec((_RB, _BH), lambda k: (k, 0)),
            pl.BlockSpec((2, _BH), lambda k: (0, 0)),
        ],
        out_shape=[
            jax.ShapeDtypeStruct((_M, _BH), jnp.float32),
            jax.ShapeDtypeStruct((2, _BH), jnp.float32),
        ],
        scratch_shapes=[pltpu.VMEM((2, _BH), jnp.float32)],
        compiler_params=pltpu.CompilerParams(vmem_limit_bytes=100 * 1024 * 1024),
    )(L, xw1, b1t)


def _bn1_body(t_ref, st_ref, g_ref, be_ref, out_ref):
    sel = _sel_fold()
    ssum = jnp.dot(st_ref[0:1, :], sel, preferred_element_type=jnp.float32,
                   precision=lax.Precision.HIGHEST)
    ssq = jnp.dot(st_ref[1:2, :], sel, preferred_element_type=jnp.float32,
                  precision=lax.Precision.HIGHEST)
    scale, shift = _scale_shift(ssum, ssq, g_ref[...], be_ref[...])
    selT = _sel_bcast(_BH)
    scale_f = jnp.dot(scale, selT, preferred_element_type=jnp.float32,
                      precision=lax.Precision.HIGHEST)
    shift_f = jnp.dot(shift, selT, preferred_element_type=jnp.float32,
                      precision=lax.Precision.HIGHEST)
    out_ref[...] = jnp.maximum(t_ref[...] * scale_f + shift_f, 0.0)


def _bn1(t1, stats, g1, be1):
    return pl.pallas_call(
        _bn1_body,
        grid=(_M // 256,),
        in_specs=[
            pl.BlockSpec((256, _BH), lambda k: (k, 0)),
            pl.BlockSpec((2, _BH), lambda k: (0, 0)),
            pl.BlockSpec((1, _HID), lambda k: (0, 0)),
            pl.BlockSpec((1, _HID), lambda k: (0, 0)),
        ],
        out_specs=pl.BlockSpec((256, _BH), lambda k: (k, 0)),
        out_shape=jax.ShapeDtypeStruct((_M, _BH), jnp.float32),
    )(t1, stats, g1, be1)


def _spmm2_body(l_ref, h_ref, out_ref):
    out_ref[...] = _dot3(l_ref[...], h_ref[...])


def _spmm2(L, h1):
    return pl.pallas_call(
        _spmm2_body,
        grid=(_M // _RB,),
        in_specs=[
            pl.BlockSpec((_RB, _M), lambda k: (k, 0)),
            pl.BlockSpec((_M, _BH), lambda k: (0, 0)),
        ],
        out_specs=pl.BlockSpec((_RB, _BH), lambda k: (k, 0)),
        out_shape=jax.ShapeDtypeStruct((_M, _BH), jnp.float32),
        compiler_params=pltpu.CompilerParams(vmem_limit_bytes=100 * 1024 * 1024),
    )(L, h1)


def _tw2_body(g_ref, w2_ref, b2_ref, out_ref, st_ref, acc_ref):
    k = pl.program_id(0)

    @pl.when(k == 0)
    def _():
        acc_ref[...] = jnp.zeros_like(acc_ref)

    gt = jnp.transpose(g_ref[...], (1, 0, 2))
    g2d = gt.reshape(_B * _RB, _HID).astype(jnp.bfloat16)
    t2 = jnp.dot(g2d, w2_ref[...],
                 preferred_element_type=jnp.float32) + b2_ref[...]
    acc_ref[0:1, :] += jnp.sum(t2, axis=0, keepdims=True)
    acc_ref[1:2, :] += jnp.sum(t2 * t2, axis=0, keepdims=True)
    out_ref[...] = t2.reshape(_B, _RB, _HID)

    @pl.when(k == pl.num_programs(0) - 1)
    def _():
        st_ref[...] = acc_ref[...]


def _tw2(g3, W2, b2):
    return pl.pallas_call(
        _tw2_body,
        grid=(_M // _RB,),
        in_specs=[
            pl.BlockSpec((_RB, _B, _HID), lambda k: (k, 0, 0)),
            pl.BlockSpec((_HID, _HID), lambda k: (0, 0)),
            pl.BlockSpec((1, _HID), lambda k: (0, 0)),
        ],
        out_specs=[
            pl.BlockSpec((_B, _RB, _HID), lambda k: (0, k, 0)),
            pl.BlockSpec((2, _HID), lambda k: (0, 0)),
        ],
        out_shape=[
            jax.ShapeDtypeStruct((_B, _M, _HID), jnp.float32),
            jax.ShapeDtypeStruct((2, _HID), jnp.float32),
        ],
        scratch_shapes=[pltpu.VMEM((2, _HID), jnp.float32)],
    )(g3, W2, b2)


_BK = 8192


def _fc_body(x_ref, st_ref, g_ref, be_ref, w_ref, ex_ref, wex_ref, b1_ref,
             w2_ref, b2_ref, out_ref, acc_ref, aff_ref):
    k = pl.program_id(0)

    @pl.when(k == 0)
    def _():
        acc_ref[...] = jnp.zeros_like(acc_ref)
        scale, shift = _scale_shift(st_ref[0:1, :], st_ref[1:2, :],
                                    g_ref[...], be_ref[...])
        selT = _sel_bcast(_BK)
        aff_ref[0:1, :] = jnp.dot(scale, selT,
                                  preferred_element_type=jnp.float32,
                                  precision=lax.Precision.HIGHEST)
        aff_ref[1:2, :] = jnp.dot(shift, selT,
                                  preferred_element_type=jnp.float32,
                                  precision=lax.Precision.HIGHEST)

    h2 = jnp.maximum(x_ref[...] * aff_ref[0:1, :] + aff_ref[1:2, :], 0.0)
    acc_ref[...] += jnp.dot(h2.astype(jnp.bfloat16), w_ref[...],
                            preferred_element_type=jnp.float32)

    @pl.when(k == pl.num_programs(0) - 1)
    def _():
        h = acc_ref[...] + jnp.dot(ex_ref[...].astype(jnp.bfloat16),
                                   wex_ref[...],
                                   preferred_element_type=jnp.float32) + b1_ref[...]
        h = jnp.maximum(h, 0.0)
        out_ref[...] = jnp.dot(h.astype(jnp.bfloat16), w2_ref[...],
                               preferred_element_type=jnp.float32) + b2_ref[...]


def _fc_head(t2, stats2, g2, be2, fc1_Wa, ex, fc1_Wb, fc1_b, fc2_W, fc2_b):
    nk = (_M * _HID) // _BK
    return pl.pallas_call(
        _fc_body,
        grid=(nk,),
        in_specs=[
            pl.BlockSpec((_B, _BK), lambda k: (0, k)),
            pl.BlockSpec((2, _HID), lambda k: (0, 0)),
            pl.BlockSpec((1, _HID), lambda k: (0, 0)),
            pl.BlockSpec((1, _HID), lambda k: (0, 0)),
            pl.BlockSpec((_BK, _FC1), lambda k: (k, 0)),
            pl.BlockSpec(ex.shape, lambda k: (0, 0)),
            pl.BlockSpec(fc1_Wb.shape, lambda k: (0, 0)),
            pl.BlockSpec((1, _FC1), lambda k: (0, 0)),
            pl.BlockSpec(fc2_W.shape, lambda k: (0, 0)),
            pl.BlockSpec(fc2_b.shape, lambda k: (0, 0)),
        ],
        out_specs=pl.BlockSpec((_B, 2), lambda k: (0, 0)),
        out_shape=jax.ShapeDtypeStruct((_B, 2), jnp.float32),
        scratch_shapes=[
            pltpu.VMEM((_B, _FC1), jnp.float32),
            pltpu.VMEM((2, _BK), jnp.float32),
        ],
        compiler_params=pltpu.CompilerParams(vmem_limit_bytes=100 * 1024 * 1024),
    )(t2, stats2, g2, be2, fc1_Wa, ex, fc1_Wb, fc1_b, fc2_W, fc2_b)


def kernel(inputs, ex_inputs, edge_index, edge_weight, W1, b1, W2, b2,
           g1, be1, g2, be2, fc1_W, fc1_b, fc2_W, fc2_b):
    ei = edge_index.astype(jnp.int32)
    edge_src = ei[0]
    edge_dst = ei[1]
    zeros = jnp.zeros((_RP * _M,), jnp.float32)
    L = _build_l(edge_dst, edge_src, edge_weight.astype(jnp.float32),
                 zeros).reshape(_M, _M)

    w1r = W1.astype(jnp.bfloat16).astype(jnp.float32)
    xw1 = _xw1(inputs, w1r)
    b1t = jnp.tile(b1, _B)[None, :]
    t1, stats1 = _spmm1(L, xw1, b1t)
    h1 = _bn1(t1, stats1, g1[None, :], be1[None, :])
    g = _spmm2(L, h1)

    t2, stats2 = _tw2(g.reshape(_M, _B, _HID), W2.astype(jnp.bfloat16),
                      b2[None, :])

    out = _fc_head(t2.reshape(_B, _M * _HID), stats2, g2[None, :],
                   be2[None, :], fc1_W[:_M * _HID].astype(jnp.bfloat16),
                   ex_inputs, fc1_W[_M * _HID:].astype(jnp.bfloat16),
                   fc1_b[None, :], fc2_W.astype(jnp.bfloat16),
                   fc2_b[None, :])
    return out
